# baseline (speedup 1.0000x reference)
import functools

import jax
import jax.numpy as jnp
from jax import lax
from jax.experimental import pallas as pl
from jax.experimental.pallas import tpu as pltpu
from jax.experimental.pallas import tpu_sc as plsc

_N = 10000
_NP = 10240
_E = 320000
_H = 64
_NC = 2
_NS = 16
_NW = _NC * _NS
_EW = _E // _NW
_C = 128
_K = 80
_EP = _NW * _K * _C
_STRIPE = _NP // _NS
_CW = 16
_NB = 4


def _sc_body(with_cnt, *refs):
    if with_cnt:
        (y_tp, y_it, ei_tp, ei_it, zeros64, zeros16,
         ones16, out_tp, out_it, out_cnt, acc, acc_cnt,
         src_tp_v, dst_tp_v, src_it_v, dst_it_v, rows_v, ones_v, sem) = refs
    else:
        (y_tp, y_it, ei_tp, ei_it, zeros64,
         out_tp, out_it, acc,
         src_tp_v, dst_tp_v, src_it_v, dst_it_v, rows_v, sem) = refs
    c = lax.axis_index("c")
    s = lax.axis_index("s")
    wid = c * _NS + s
    stripe = s * _STRIPE

    pltpu.sync_copy(zeros64.at[pl.ds(stripe, _STRIPE)],
                    acc.at[pl.ds(stripe, _STRIPE)])
    pltpu.sync_copy(ei_tp.at[0].at[wid], src_tp_v)
    pltpu.sync_copy(ei_tp.at[1].at[wid], dst_tp_v)
    pltpu.sync_copy(ei_it.at[0].at[wid], src_it_v)
    pltpu.sync_copy(ei_it.at[1].at[wid], dst_it_v)
    if with_cnt:
        pltpu.sync_copy(zeros16.at[pl.ds(stripe, _STRIPE)],
                        acc_cnt.at[pl.ds(stripe, _STRIPE)])
        pltpu.sync_copy(ones16, ones_v)
    plsc.subcore_barrier()

    def run_type(y, src_v, dst_v, count):
        for b in range(_NB):
            pltpu.async_copy(y.at[src_v.at[b]], rows_v[b], sem)

        def group(g, prefetch):
            for b in range(_NB):
                j = g * _NB + b
                pltpu.make_async_copy(y.at[src_v.at[j]], rows_v[b],
                                      sem).wait()
                pltpu.sync_copy(rows_v[b], acc.at[dst_v.at[j]], add=True)
                if count:
                    pltpu.sync_copy(ones_v, acc_cnt.at[dst_v.at[j]],
                                    add=True)
                if prefetch:
                    pltpu.async_copy(y.at[src_v.at[j + _NB]], rows_v[b], sem)

        lax.fori_loop(0, _K // _NB - 1, lambda g, _: (group(g, True), 0)[1],
                      0)
        group(_K // _NB - 1, False)

    run_type(y_tp, src_tp_v, dst_tp_v, False)
    plsc.subcore_barrier()
    pltpu.sync_copy(acc.at[pl.ds(stripe, _STRIPE)],
                    out_tp.at[c].at[pl.ds(stripe, _STRIPE)])
    pltpu.sync_copy(zeros64.at[pl.ds(stripe, _STRIPE)],
                    acc.at[pl.ds(stripe, _STRIPE)])
    plsc.subcore_barrier()

    run_type(y_it, src_it_v, dst_it_v, with_cnt)
    plsc.subcore_barrier()
    pltpu.sync_copy(acc.at[pl.ds(stripe, _STRIPE)],
                    out_it.at[c].at[pl.ds(stripe, _STRIPE)])
    if with_cnt:
        pltpu.sync_copy(acc_cnt.at[pl.ds(stripe, _STRIPE)],
                        out_cnt.at[c].at[pl.ds(stripe, _STRIPE)])


def _make_scatter(with_cnt):
    mesh = plsc.VectorSubcoreMesh(core_axis_name="c", subcore_axis_name="s")
    params = pltpu.CompilerParams(use_tc_tiling_on_sc=False)
    out_type = [jax.ShapeDtypeStruct((_NC, _NP, _H), jnp.float32),
                jax.ShapeDtypeStruct((_NC, _NP, _H), jnp.float32)]
    scratch = [
        pltpu.VMEM_SHARED((_NP, _H), jnp.float32),
        pltpu.VMEM((_K, _C), jnp.int32),
        pltpu.VMEM((_K, _C), jnp.int32),
        pltpu.VMEM((_K, _C), jnp.int32),
        pltpu.VMEM((_K, _C), jnp.int32),
        [pltpu.VMEM((_C, _H), jnp.float32)
         for _ in range(_NB)],
        pltpu.SemaphoreType.DMA,
    ]
    if with_cnt:
        out_type = out_type + [
            jax.ShapeDtypeStruct((_NC, _NP, _CW), jnp.float32)]
        scratch = ([pltpu.VMEM_SHARED((_NP, _CW), jnp.float32)]
                   + scratch[1:] if False else
                   [scratch[0], pltpu.VMEM_SHARED((_NP, _CW), jnp.float32)]
                   + scratch[1:5]
                   + [scratch[5], pltpu.VMEM((_C, _CW), jnp.float32),
                      scratch[6]])
    return pl.kernel(functools.partial(_sc_body, with_cnt),
                     out_type=out_type, mesh=mesh, scratch_types=scratch,
                     compiler_params=params,
                     name="sc_scatter_cnt" if with_cnt else "sc_scatter")


_sc_scatter_cnt = _make_scatter(True)
_sc_scatter = _make_scatter(False)


def _head_body(x_ref, w_ref, ytp_ref, yit_ref):
    y = jnp.dot(x_ref[...], w_ref[...], preferred_element_type=jnp.float32)
    ytp_ref[...] = y[:, :_H]
    yit_ref[...] = y[:, _H:]


def _head_mm(x, w_cat):
    return pl.pallas_call(
        _head_body,
        out_shape=[jax.ShapeDtypeStruct((_NP, _H), jnp.float32),
                   jax.ShapeDtypeStruct((_NP, _H), jnp.float32)],
    )(x, w_cat)


def _combine_body(residual, ptp_ref, pit_ref, cnt_ref, b_ref, xprev_ref,
                  w_ref, x_ref, ytp_ref, yit_ref):
    cnt = cnt_ref[0] + cnt_ref[1]
    inv = 1.0 / jnp.maximum(cnt[:, 0:1], 1.0)
    agg = (ptp_ref[0] + ptp_ref[1]
           + (pit_ref[0] + pit_ref[1]) * inv
           + b_ref[...])
    x = jnp.maximum(agg, 0.0)
    if residual:
        x = x + xprev_ref[...]
    x_ref[...] = x
    y = jnp.dot(x, w_ref[...], preferred_element_type=jnp.float32)
    ytp_ref[...] = y[:, :_H]
    yit_ref[...] = y[:, _H:]


def _combine_mm(ptp, pit, cnt, b, xprev, w_cat, residual):
    return pl.pallas_call(
        functools.partial(_combine_body, residual),
        out_shape=[jax.ShapeDtypeStruct((_NP, _H), jnp.float32),
                   jax.ShapeDtypeStruct((_NP, _H), jnp.float32),
                   jax.ShapeDtypeStruct((_NP, _H), jnp.float32)],
    )(ptp, pit, cnt, b, xprev, w_cat)


def _tail_body(ptp_ref, pit_ref, cnt_ref, b_ref, xprev_ref,
               wl_ref, bl_ref, wd1_ref, bd1_ref, wd2_ref, bd2_ref, out_ref):
    cnt = cnt_ref[0] + cnt_ref[1]
    inv = 1.0 / jnp.maximum(cnt[:, 0:1], 1.0)
    agg = (ptp_ref[0] + ptp_ref[1]
           + (pit_ref[0] + pit_ref[1]) * inv
           + b_ref[...])
    x = jnp.maximum(agg, 0.0) + xprev_ref[...]
    feat = jnp.dot(x, wl_ref[...], preferred_element_type=jnp.float32)
    feat = feat + bl_ref[...]
    h = jnp.maximum(
        jnp.dot(feat, wd1_ref[...], preferred_element_type=jnp.float32)
        + bd1_ref[...], 0.0)
    logits = (jnp.dot(h, wd2_ref[...], preferred_element_type=jnp.float32)
              + bd2_ref[...])
    out_ref[...] = jax.nn.sigmoid(logits)


def _tail(ptp, pit, cnt, b, xprev, wl, bl, wd1, bd1, wd2, bd2):
    return pl.pallas_call(
        _tail_body,
        out_shape=jax.ShapeDtypeStruct((_NP, 9), jnp.float32),
    )(ptp, pit, cnt, b, xprev, wl, bl, wd1, bd1, wd2, bd2)


def kernel(x_stroke, edge_index_temp_previous, edge_index_intersects,
           W_head_tp, W_head_int, b_head,
           W_tp1, W_int1, b1, W_tp2, W_int2, b2,
           W_tp3, W_int3, b3, W_tp4, W_int4, b4,
           Wl, bl, Wd1, bd1, Wd2, bd2):
    pad_src = jnp.full((1, _EP - _E), _N, jnp.int32)
    pad_dst = (_N + jnp.arange(_EP - _E, dtype=jnp.int32) % (_NP - _N))[None]
    pad_ei = jnp.concatenate([pad_src, pad_dst], axis=0)
    ei_tp = jnp.concatenate([edge_index_temp_previous, pad_ei],
                            axis=1).reshape(2, _NW, _K, _C)
    ei_it = jnp.concatenate([edge_index_intersects, pad_ei],
                            axis=1).reshape(2, _NW, _K, _C)

    zeros64 = jnp.zeros((_NP, _H), jnp.float32)
    zeros16 = jnp.zeros((_NP, _CW), jnp.float32)
    ones16 = jnp.ones((_C, _CW), jnp.float32)

    w_head_cat = jnp.concatenate([W_head_tp, W_head_int], axis=1)
    w_cats = [jnp.concatenate([wt, wi], axis=1)
              for (wt, wi) in ((W_tp1, W_int1), (W_tp2, W_int2),
                               (W_tp3, W_int3), (W_tp4, W_int4))]
    biases = [b_head.reshape(1, _H), b1.reshape(1, _H), b2.reshape(1, _H),
              b3.reshape(1, _H), b4.reshape(1, _H)]

    x_pad = jnp.pad(x_stroke, ((0, _NP - _N), (0, 0)))
    y_tp, y_it = _head_mm(x_pad, w_head_cat)

    ptp, pit, cntp = _sc_scatter_cnt(y_tp, y_it, ei_tp, ei_it,
                                     zeros64, zeros16, ones16)

    xprev = jnp.zeros((_NP, _H), jnp.float32)
    x, y_tp, y_it = _combine_mm(ptp, pit, cntp, biases[0], xprev,
                                w_cats[0], residual=False)

    for layer in range(1, 5):
        ptp, pit = _sc_scatter(y_tp, y_it, ei_tp, ei_it, zeros64)
        if layer < 4:
            xnew, y_tp, y_it = _combine_mm(ptp, pit, cntp, biases[layer], x,
                                           w_cats[layer], residual=True)
            x = xnew
        else:
            out = _tail(ptp, pit, cntp, biases[4], x,
                        Wl, bl.reshape(1, 128), Wd1, bd1.reshape(1, _H),
                        Wd2, bd2.reshape(1, 9))
            return out[:_N]


# speedup vs baseline: 3.7646x; 3.7646x over previous
import functools

import jax
import jax.numpy as jnp
from jax import lax
from jax.experimental import pallas as pl
from jax.experimental.pallas import tpu as pltpu
from jax.experimental.pallas import tpu_sc as plsc

_N = 10000
_NP = 10240
_E = 320000
_H = 64
_NC = 2
_NS = 16
_NW = _NC * _NS
_EW = _E // _NW
_C = 125
_K = 80
_EP = _NW * _K * _C
_STRIPE = _NP // _NS
_CW = 16
_NB = 4


def _sc_body(with_cnt, *refs):
    if with_cnt:
        (y_tp, y_it, ei_tp, ei_it, zeros64, zeros16,
         ones16, out_tp, out_it, out_cnt, acc, acc_cnt,
         src_tp_v, dst_tp_v, src_it_v, dst_it_v, rows_v, ones_v, sem) = refs
    else:
        (y_tp, y_it, ei_tp, ei_it, zeros64,
         out_tp, out_it, acc,
         src_tp_v, dst_tp_v, src_it_v, dst_it_v, rows_v, sem) = refs
    c = lax.axis_index("c")
    s = lax.axis_index("s")
    wid = c * _NS + s
    stripe = s * _STRIPE

    pltpu.sync_copy(zeros64.at[pl.ds(stripe, _STRIPE)],
                    acc.at[pl.ds(stripe, _STRIPE)])
    pltpu.sync_copy(ei_tp.at[0].at[wid], src_tp_v)
    pltpu.sync_copy(ei_tp.at[1].at[wid], dst_tp_v)
    pltpu.sync_copy(ei_it.at[0].at[wid], src_it_v)
    pltpu.sync_copy(ei_it.at[1].at[wid], dst_it_v)
    if with_cnt:
        pltpu.sync_copy(zeros16.at[pl.ds(stripe, _STRIPE)],
                        acc_cnt.at[pl.ds(stripe, _STRIPE)])
        pltpu.sync_copy(ones16, ones_v)
    plsc.subcore_barrier()

    def run_type(y, src_v, dst_v, count):
        for b in range(_NB):
            pltpu.async_copy(y.at[src_v.at[b]], rows_v[b], sem)

        def group(g, prefetch):
            for b in range(_NB):
                j = g * _NB + b
                pltpu.make_async_copy(y.at[src_v.at[j]], rows_v[b],
                                      sem).wait()
                pltpu.sync_copy(rows_v[b], acc.at[dst_v.at[j]], add=True)
                if count:
                    pltpu.sync_copy(ones_v, acc_cnt.at[dst_v.at[j]],
                                    add=True)
                if prefetch:
                    pltpu.async_copy(y.at[src_v.at[j + _NB]], rows_v[b], sem)

        lax.fori_loop(0, _K // _NB - 1, lambda g, _: (group(g, True), 0)[1],
                      0)
        group(_K // _NB - 1, False)

    run_type(y_tp, src_tp_v, dst_tp_v, False)
    plsc.subcore_barrier()
    pltpu.sync_copy(acc.at[pl.ds(stripe, _STRIPE)],
                    out_tp.at[c].at[pl.ds(stripe, _STRIPE)])
    pltpu.sync_copy(zeros64.at[pl.ds(stripe, _STRIPE)],
                    acc.at[pl.ds(stripe, _STRIPE)])
    plsc.subcore_barrier()

    run_type(y_it, src_it_v, dst_it_v, with_cnt)
    plsc.subcore_barrier()
    pltpu.sync_copy(acc.at[pl.ds(stripe, _STRIPE)],
                    out_it.at[c].at[pl.ds(stripe, _STRIPE)])
    if with_cnt:
        pltpu.sync_copy(acc_cnt.at[pl.ds(stripe, _STRIPE)],
                        out_cnt.at[c].at[pl.ds(stripe, _STRIPE)])


def _make_scatter(with_cnt):
    mesh = plsc.VectorSubcoreMesh(core_axis_name="c", subcore_axis_name="s")
    params = pltpu.CompilerParams(use_tc_tiling_on_sc=False)
    out_type = [jax.ShapeDtypeStruct((_NC, _NP, _H), jnp.float32),
                jax.ShapeDtypeStruct((_NC, _NP, _H), jnp.float32)]
    scratch = [
        pltpu.VMEM_SHARED((_NP, _H), jnp.float32),
        pltpu.VMEM((_K, _C), jnp.int32),
        pltpu.VMEM((_K, _C), jnp.int32),
        pltpu.VMEM((_K, _C), jnp.int32),
        pltpu.VMEM((_K, _C), jnp.int32),
        [pltpu.VMEM((_C, _H), jnp.float32)
         for _ in range(_NB)],
        pltpu.SemaphoreType.DMA,
    ]
    if with_cnt:
        out_type = out_type + [
            jax.ShapeDtypeStruct((_NC, _NP, _CW), jnp.float32)]
        scratch = ([pltpu.VMEM_SHARED((_NP, _CW), jnp.float32)]
                   + scratch[1:] if False else
                   [scratch[0], pltpu.VMEM_SHARED((_NP, _CW), jnp.float32)]
                   + scratch[1:5]
                   + [scratch[5], pltpu.VMEM((_C, _CW), jnp.float32),
                      scratch[6]])
    return pl.kernel(functools.partial(_sc_body, with_cnt),
                     out_type=out_type, mesh=mesh, scratch_types=scratch,
                     compiler_params=params,
                     name="sc_scatter_cnt" if with_cnt else "sc_scatter")


_sc_scatter_cnt = _make_scatter(True)
_sc_scatter = _make_scatter(False)


def _head_body(x_ref, w_ref, ytp_ref, yit_ref):
    y = jnp.dot(x_ref[...], w_ref[...], preferred_element_type=jnp.float32)
    ytp_ref[...] = y[:, :_H]
    yit_ref[...] = y[:, _H:]


def _head_mm(x, w_cat):
    return pl.pallas_call(
        _head_body,
        out_shape=[jax.ShapeDtypeStruct((_NP, _H), jnp.float32),
                   jax.ShapeDtypeStruct((_NP, _H), jnp.float32)],
    )(x, w_cat)


def _combine_body(residual, ptp_ref, pit_ref, cnt_ref, b_ref, xprev_ref,
                  w_ref, x_ref, ytp_ref, yit_ref):
    cnt = cnt_ref[0] + cnt_ref[1]
    inv = 1.0 / jnp.maximum(cnt[:, 0:1], 1.0)
    agg = (ptp_ref[0] + ptp_ref[1]
           + (pit_ref[0] + pit_ref[1]) * inv
           + b_ref[...])
    x = jnp.maximum(agg, 0.0)
    if residual:
        x = x + xprev_ref[...]
    x_ref[...] = x
    y = jnp.dot(x, w_ref[...], preferred_element_type=jnp.float32)
    ytp_ref[...] = y[:, :_H]
    yit_ref[...] = y[:, _H:]


def _combine_mm(ptp, pit, cnt, b, xprev, w_cat, residual):
    return pl.pallas_call(
        functools.partial(_combine_body, residual),
        out_shape=[jax.ShapeDtypeStruct((_NP, _H), jnp.float32),
                   jax.ShapeDtypeStruct((_NP, _H), jnp.float32),
                   jax.ShapeDtypeStruct((_NP, _H), jnp.float32)],
    )(ptp, pit, cnt, b, xprev, w_cat)


def _tail_body(ptp_ref, pit_ref, cnt_ref, b_ref, xprev_ref,
               wl_ref, bl_ref, wd1_ref, bd1_ref, wd2_ref, bd2_ref, out_ref):
    cnt = cnt_ref[0] + cnt_ref[1]
    inv = 1.0 / jnp.maximum(cnt[:, 0:1], 1.0)
    agg = (ptp_ref[0] + ptp_ref[1]
           + (pit_ref[0] + pit_ref[1]) * inv
           + b_ref[...])
    x = jnp.maximum(agg, 0.0) + xprev_ref[...]
    feat = jnp.dot(x, wl_ref[...], preferred_element_type=jnp.float32)
    feat = feat + bl_ref[...]
    h = jnp.maximum(
        jnp.dot(feat, wd1_ref[...], preferred_element_type=jnp.float32)
        + bd1_ref[...], 0.0)
    logits = (jnp.dot(h, wd2_ref[...], preferred_element_type=jnp.float32)
              + bd2_ref[...])
    out_ref[...] = jax.nn.sigmoid(logits)


def _tail(ptp, pit, cnt, b, xprev, wl, bl, wd1, bd1, wd2, bd2):
    return pl.pallas_call(
        _tail_body,
        out_shape=jax.ShapeDtypeStruct((_NP, 9), jnp.float32),
    )(ptp, pit, cnt, b, xprev, wl, bl, wd1, bd1, wd2, bd2)


def kernel(x_stroke, edge_index_temp_previous, edge_index_intersects,
           W_head_tp, W_head_int, b_head,
           W_tp1, W_int1, b1, W_tp2, W_int2, b2,
           W_tp3, W_int3, b3, W_tp4, W_int4, b4,
           Wl, bl, Wd1, bd1, Wd2, bd2):
    ei_tp = edge_index_temp_previous.reshape(2, _NW, _K, _C)
    ei_it = edge_index_intersects.reshape(2, _NW, _K, _C)

    zeros64 = jnp.zeros((_NP, _H), jnp.float32)
    zeros16 = jnp.zeros((_NP, _CW), jnp.float32)
    ones16 = jnp.ones((_C, _CW), jnp.float32)

    w_head_cat = jnp.concatenate([W_head_tp, W_head_int], axis=1)
    w_cats = [jnp.concatenate([wt, wi], axis=1)
              for (wt, wi) in ((W_tp1, W_int1), (W_tp2, W_int2),
                               (W_tp3, W_int3), (W_tp4, W_int4))]
    biases = [b_head.reshape(1, _H), b1.reshape(1, _H), b2.reshape(1, _H),
              b3.reshape(1, _H), b4.reshape(1, _H)]

    x_pad = jnp.pad(x_stroke, ((0, _NP - _N), (0, 0)))
    y_tp, y_it = _head_mm(x_pad, w_head_cat)

    ptp, pit, cntp = _sc_scatter_cnt(y_tp, y_it, ei_tp, ei_it,
                                     zeros64, zeros16, ones16)

    xprev = jnp.zeros((_NP, _H), jnp.float32)
    x, y_tp, y_it = _combine_mm(ptp, pit, cntp, biases[0], xprev,
                                w_cats[0], residual=False)

    for layer in range(1, 5):
        ptp, pit = _sc_scatter(y_tp, y_it, ei_tp, ei_it, zeros64)
        if layer < 4:
            xnew, y_tp, y_it = _combine_mm(ptp, pit, cntp, biases[layer], x,
                                           w_cats[layer], residual=True)
            x = xnew
        else:
            out = _tail(ptp, pit, cntp, biases[4], x,
                        Wl, bl.reshape(1, 128), Wd1, bd1.reshape(1, _H),
                        Wd2, bd2.reshape(1, 9))
            return out[:_N]


# speedup vs baseline: 4.2219x; 1.1214x over previous
import functools

import jax
import jax.numpy as jnp
from jax import lax
from jax.experimental import pallas as pl
from jax.experimental.pallas import tpu as pltpu
from jax.experimental.pallas import tpu_sc as plsc

_N = 10000
_NP = 10240
_E = 320000
_H = 64
_NC = 2
_NS = 16
_NW = _NC * _NS
_EW = _E // _NW
_C = 125
_K = 80
_EP = _NW * _K * _C
_STRIPE = _NP // _NS
_CW = 16
_NB = 4


def _sc_body(with_cnt, *refs):
    if with_cnt:
        (y_tp, y_it, ei_tp, ei_it, zeros64, zeros16,
         ones16, out_pp, out_cnt, acc, acc_cnt,
         src_tp_v, dst_tp_v, src_it_v, dst_it_v, rows_v, ones_v, sem) = refs
    else:
        (y_tp, y_it, ei_tp, ei_it, zeros64,
         out_pp, acc,
         src_tp_v, dst_tp_v, src_it_v, dst_it_v, rows_v, sem) = refs
    c = lax.axis_index("c")
    s = lax.axis_index("s")
    wid = c * _NS + s
    stripe = s * _STRIPE

    pltpu.sync_copy(zeros64.at[pl.ds(stripe, _STRIPE)],
                    acc.at[pl.ds(stripe, _STRIPE)])
    pltpu.sync_copy(ei_tp.at[0].at[wid], src_tp_v)
    pltpu.sync_copy(ei_tp.at[1].at[wid], dst_tp_v)
    pltpu.sync_copy(ei_it.at[0].at[wid], src_it_v)
    pltpu.sync_copy(ei_it.at[1].at[wid], dst_it_v)
    if with_cnt:
        pltpu.sync_copy(zeros16.at[pl.ds(stripe, _STRIPE)],
                        acc_cnt.at[pl.ds(stripe, _STRIPE)])
        pltpu.sync_copy(ones16, ones_v)
    plsc.subcore_barrier()

    def run_type(y, src_v, dst_v, count):
        for b in range(_NB):
            pltpu.async_copy(y.at[src_v.at[b]], rows_v[b], sem)

        def group(g, prefetch):
            for b in range(_NB):
                j = g * _NB + b
                pltpu.make_async_copy(y.at[src_v.at[j]], rows_v[b],
                                      sem).wait()
                pltpu.sync_copy(rows_v[b], acc.at[dst_v.at[j]], add=True)
                if count:
                    pltpu.sync_copy(ones_v, acc_cnt.at[dst_v.at[j]],
                                    add=True)
                if prefetch:
                    pltpu.async_copy(y.at[src_v.at[j + _NB]], rows_v[b], sem)

        lax.fori_loop(0, _K // _NB - 1, lambda g, _: (group(g, True), 0)[1],
                      0)
        group(_K // _NB - 1, False)

    run_type(y_tp, src_tp_v, dst_tp_v, False)
    plsc.subcore_barrier()
    pltpu.sync_copy(acc.at[pl.ds(stripe, _STRIPE)],
                    out_pp.at[c].at[pl.ds(stripe, _STRIPE), pl.ds(0, _H)])
    pltpu.sync_copy(zeros64.at[pl.ds(stripe, _STRIPE)],
                    acc.at[pl.ds(stripe, _STRIPE)])
    plsc.subcore_barrier()

    run_type(y_it, src_it_v, dst_it_v, with_cnt)
    plsc.subcore_barrier()
    pltpu.sync_copy(acc.at[pl.ds(stripe, _STRIPE)],
                    out_pp.at[c].at[pl.ds(stripe, _STRIPE), pl.ds(_H, _H)])
    if with_cnt:
        pltpu.sync_copy(acc_cnt.at[pl.ds(stripe, _STRIPE)],
                        out_cnt.at[c].at[pl.ds(stripe, _STRIPE)])


def _make_scatter(with_cnt):
    mesh = plsc.VectorSubcoreMesh(core_axis_name="c", subcore_axis_name="s")
    params = pltpu.CompilerParams(use_tc_tiling_on_sc=False)
    out_type = [jax.ShapeDtypeStruct((_NC, _NP, 2 * _H), jnp.float32)]
    scratch = [
        pltpu.VMEM_SHARED((_NP, _H), jnp.float32),
        pltpu.VMEM((_K, _C), jnp.int32),
        pltpu.VMEM((_K, _C), jnp.int32),
        pltpu.VMEM((_K, _C), jnp.int32),
        pltpu.VMEM((_K, _C), jnp.int32),
        [pltpu.VMEM((_C, _H), jnp.float32)
         for _ in range(_NB)],
        pltpu.SemaphoreType.DMA,
    ]
    if with_cnt:
        out_type = out_type + [
            jax.ShapeDtypeStruct((_NC, _NP, _CW), jnp.float32)]
        scratch = ([pltpu.VMEM_SHARED((_NP, _CW), jnp.float32)]
                   + scratch[1:] if False else
                   [scratch[0], pltpu.VMEM_SHARED((_NP, _CW), jnp.float32)]
                   + scratch[1:5]
                   + [scratch[5], pltpu.VMEM((_C, _CW), jnp.float32),
                      scratch[6]])
    return pl.kernel(functools.partial(_sc_body, with_cnt),
                     out_type=out_type, mesh=mesh, scratch_types=scratch,
                     compiler_params=params,
                     name="sc_scatter_cnt" if with_cnt else "sc_scatter")


_sc_scatter_cnt = _make_scatter(True)
_sc_scatter = _make_scatter(False)


def _head_body(x_ref, w_ref, ytp_ref, yit_ref):
    y = jnp.dot(x_ref[...], w_ref[...], preferred_element_type=jnp.float32)
    ytp_ref[...] = y[:, :_H]
    yit_ref[...] = y[:, _H:]


def _head_mm(x, w_cat):
    return pl.pallas_call(
        _head_body,
        out_shape=[jax.ShapeDtypeStruct((_NP, _H), jnp.float32),
                   jax.ShapeDtypeStruct((_NP, _H), jnp.float32)],
    )(x, w_cat)


def _combine_body(residual, pp_ref, cnt_ref, b_ref, xprev_ref,
                  w_ref, x_ref, ytp_ref, yit_ref):
    cnt = cnt_ref[0] + cnt_ref[1]
    inv = 1.0 / jnp.maximum(cnt[:, 0:1], 1.0)
    pp = pp_ref[0] + pp_ref[1]
    agg = pp[:, :_H] + pp[:, _H:] * inv + b_ref[...]
    x = jnp.maximum(agg, 0.0)
    if residual:
        x = x + xprev_ref[...]
    x_ref[...] = x
    y = jnp.dot(x, w_ref[...], preferred_element_type=jnp.float32)
    ytp_ref[...] = y[:, :_H]
    yit_ref[...] = y[:, _H:]


def _combine_mm(pp, cnt, b, xprev, w_cat, residual):
    return pl.pallas_call(
        functools.partial(_combine_body, residual),
        out_shape=[jax.ShapeDtypeStruct((_NP, _H), jnp.float32),
                   jax.ShapeDtypeStruct((_NP, _H), jnp.float32),
                   jax.ShapeDtypeStruct((_NP, _H), jnp.float32)],
    )(pp, cnt, b, xprev, w_cat)


def _tail_body(pp_ref, cnt_ref, b_ref, xprev_ref,
               wl_ref, bl_ref, wd1_ref, bd1_ref, wd2_ref, bd2_ref, out_ref):
    cnt = cnt_ref[0] + cnt_ref[1]
    inv = 1.0 / jnp.maximum(cnt[:, 0:1], 1.0)
    pp = pp_ref[0] + pp_ref[1]
    agg = pp[:, :_H] + pp[:, _H:] * inv + b_ref[...]
    x = jnp.maximum(agg, 0.0) + xprev_ref[...]
    feat = jnp.dot(x, wl_ref[...], preferred_element_type=jnp.float32)
    feat = feat + bl_ref[...]
    h = jnp.maximum(
        jnp.dot(feat, wd1_ref[...], preferred_element_type=jnp.float32)
        + bd1_ref[...], 0.0)
    logits = (jnp.dot(h, wd2_ref[...], preferred_element_type=jnp.float32)
              + bd2_ref[...])
    out_ref[...] = jax.nn.sigmoid(logits)


def _tail(pp, cnt, b, xprev, wl, bl, wd1, bd1, wd2, bd2):
    return pl.pallas_call(
        _tail_body,
        out_shape=jax.ShapeDtypeStruct((_NP, 9), jnp.float32),
    )(pp, cnt, b, xprev, wl, bl, wd1, bd1, wd2, bd2)


def kernel(x_stroke, edge_index_temp_previous, edge_index_intersects,
           W_head_tp, W_head_int, b_head,
           W_tp1, W_int1, b1, W_tp2, W_int2, b2,
           W_tp3, W_int3, b3, W_tp4, W_int4, b4,
           Wl, bl, Wd1, bd1, Wd2, bd2):
    ei_tp = edge_index_temp_previous.reshape(2, _NW, _K, _C)
    ei_it = edge_index_intersects.reshape(2, _NW, _K, _C)

    zeros64 = jnp.zeros((_NP, _H), jnp.float32)
    zeros16 = jnp.zeros((_NP, _CW), jnp.float32)
    ones16 = jnp.ones((_C, _CW), jnp.float32)

    w_head_cat = jnp.concatenate([W_head_tp, W_head_int], axis=1)
    w_cats = [jnp.concatenate([wt, wi], axis=1)
              for (wt, wi) in ((W_tp1, W_int1), (W_tp2, W_int2),
                               (W_tp3, W_int3), (W_tp4, W_int4))]
    biases = [b_head.reshape(1, _H), b1.reshape(1, _H), b2.reshape(1, _H),
              b3.reshape(1, _H), b4.reshape(1, _H)]

    x_pad = jnp.pad(x_stroke, ((0, _NP - _N), (0, 0)))
    y_tp, y_it = _head_mm(x_pad, w_head_cat)

    pp, cntp = _sc_scatter_cnt(y_tp, y_it, ei_tp, ei_it,
                               zeros64, zeros16, ones16)

    xprev = jnp.zeros((_NP, _H), jnp.float32)
    x, y_tp, y_it = _combine_mm(pp, cntp, biases[0], xprev,
                                w_cats[0], residual=False)

    for layer in range(1, 5):
        (pp,) = _sc_scatter(y_tp, y_it, ei_tp, ei_it, zeros64)
        if layer < 4:
            x, y_tp, y_it = _combine_mm(pp, cntp, biases[layer], x,
                                        w_cats[layer], residual=True)
        else:
            out = _tail(pp, cntp, biases[4], x,
                        Wl, bl.reshape(1, 128), Wd1, bd1.reshape(1, _H),
                        Wd2, bd2.reshape(1, 9))
            return out[:_N]


# speedup vs baseline: 4.2268x; 1.0012x over previous
import functools

import jax
import jax.numpy as jnp
from jax import lax
from jax.experimental import pallas as pl
from jax.experimental.pallas import tpu as pltpu
from jax.experimental.pallas import tpu_sc as plsc

_N = 10000
_NP = 10240
_E = 320000
_H = 64
_NC = 2
_NS = 16
_NW = _NC * _NS
_EW = _E // _NW
_C = 125
_K = 80
_EP = _NW * _K * _C
_STRIPE = _NP // _NS
_CW = 16
_NB = 4


def _sc_body(with_cnt, *refs):
    (y_tp, y_it, ei_tp, ei_it, zeros64,
     out_pp, acc,
     src_tp_v, dst_tp_v, src_it_v, dst_it_v, rows_v, sem) = refs
    c = lax.axis_index("c")
    s = lax.axis_index("s")
    wid = c * _NS + s
    stripe = s * _STRIPE

    pltpu.sync_copy(zeros64, acc.at[pl.ds(stripe, _STRIPE)])
    pltpu.sync_copy(ei_tp.at[0].at[wid], src_tp_v)
    pltpu.sync_copy(ei_tp.at[1].at[wid], dst_tp_v)
    pltpu.sync_copy(ei_it.at[0].at[wid], src_it_v)
    pltpu.sync_copy(ei_it.at[1].at[wid], dst_it_v)
    plsc.subcore_barrier()

    def run_type(y, src_v, dst_v):
        for b in range(_NB):
            pltpu.async_copy(y.at[src_v.at[b]], rows_v[b], sem)

        def group(g, prefetch):
            for b in range(_NB):
                j = g * _NB + b
                pltpu.make_async_copy(y.at[src_v.at[j]], rows_v[b],
                                      sem).wait()
                pltpu.sync_copy(rows_v[b], acc.at[dst_v.at[j]], add=True)
                if prefetch:
                    pltpu.async_copy(y.at[src_v.at[j + _NB]], rows_v[b], sem)

        lax.fori_loop(0, _K // _NB - 1, lambda g, _: (group(g, True), 0)[1],
                      0)
        group(_K // _NB - 1, False)

    run_type(y_tp, src_tp_v, dst_tp_v)
    plsc.subcore_barrier()
    pltpu.sync_copy(acc.at[pl.ds(stripe, _STRIPE)],
                    out_pp.at[c].at[pl.ds(stripe, _STRIPE), pl.ds(0, _H)])
    pltpu.sync_copy(zeros64, acc.at[pl.ds(stripe, _STRIPE)])
    plsc.subcore_barrier()

    run_type(y_it, src_it_v, dst_it_v)
    plsc.subcore_barrier()
    pltpu.sync_copy(acc.at[pl.ds(stripe, _STRIPE)],
                    out_pp.at[c].at[pl.ds(stripe, _STRIPE), pl.ds(_H, _H)])


def _cnt_body(ei_it, zeros16, ones16, out_cnt, acc_cnt, dst_it_v, ones_v):
    c = lax.axis_index("c")
    s = lax.axis_index("s")
    wid = c * _NS + s
    stripe = s * _STRIPE
    pltpu.sync_copy(zeros16, acc_cnt.at[pl.ds(stripe, _STRIPE)])
    pltpu.sync_copy(ei_it.at[1].at[wid], dst_it_v)
    pltpu.sync_copy(ones16, ones_v)
    plsc.subcore_barrier()

    def chunk(j, _):
        pltpu.sync_copy(ones_v, acc_cnt.at[dst_it_v.at[j]], add=True)
        return 0

    lax.fori_loop(0, _K, chunk, 0)
    plsc.subcore_barrier()
    pltpu.sync_copy(acc_cnt.at[pl.ds(stripe, _STRIPE)],
                    out_cnt.at[c].at[pl.ds(stripe, _STRIPE)])


def _make_kernels():
    mesh = plsc.VectorSubcoreMesh(core_axis_name="c", subcore_axis_name="s")
    params = pltpu.CompilerParams(use_tc_tiling_on_sc=False)
    scatter = pl.kernel(
        functools.partial(_sc_body, False),
        out_type=[jax.ShapeDtypeStruct((_NC, _NP, 2 * _H), jnp.float32)],
        mesh=mesh,
        scratch_types=[
            pltpu.VMEM_SHARED((_NP, _H), jnp.float32),
            pltpu.VMEM((_K, _C), jnp.int32),
            pltpu.VMEM((_K, _C), jnp.int32),
            pltpu.VMEM((_K, _C), jnp.int32),
            pltpu.VMEM((_K, _C), jnp.int32),
            [pltpu.VMEM((_C, _H), jnp.float32)
             for _ in range(_NB)],
            pltpu.SemaphoreType.DMA,
        ],
        compiler_params=params, name="sc_scatter")
    cnt = pl.kernel(
        _cnt_body,
        out_type=[jax.ShapeDtypeStruct((_NC, _NP, _CW), jnp.float32)],
        mesh=mesh,
        scratch_types=[
            pltpu.VMEM_SHARED((_NP, _CW), jnp.float32),
            pltpu.VMEM((_K, _C), jnp.int32),
            pltpu.VMEM((_C, _CW), jnp.float32),
        ],
        compiler_params=params, name="sc_count")
    return scatter, cnt


_sc_scatter, _sc_count = _make_kernels()


def _head_body(x_ref, w_ref, ytp_ref, yit_ref):
    y = jnp.dot(x_ref[...], w_ref[...], preferred_element_type=jnp.float32)
    ytp_ref[pl.ds(0, _N)] = y[:, :_H]
    yit_ref[pl.ds(0, _N)] = y[:, _H:]
    pad = jnp.zeros((_NP - _N, _H), jnp.float32)
    ytp_ref[pl.ds(_N, _NP - _N)] = pad
    yit_ref[pl.ds(_N, _NP - _N)] = pad


def _head_mm(x, w_cat):
    return pl.pallas_call(
        _head_body,
        out_shape=[jax.ShapeDtypeStruct((_NP, _H), jnp.float32),
                   jax.ShapeDtypeStruct((_NP, _H), jnp.float32)],
    )(x, w_cat)


def _combine_body(residual, pp_ref, cnt_ref, b_ref, xprev_ref,
                  w_ref, x_ref, ytp_ref, yit_ref):
    cnt = cnt_ref[0] + cnt_ref[1]
    inv = 1.0 / jnp.maximum(cnt[:, 0:1], 1.0)
    pp = pp_ref[0] + pp_ref[1]
    agg = pp[:, :_H] + pp[:, _H:] * inv + b_ref[...]
    x = jnp.maximum(agg, 0.0)
    if residual:
        x = x + xprev_ref[...]
    x_ref[...] = x
    y = jnp.dot(x, w_ref[...], preferred_element_type=jnp.float32)
    ytp_ref[...] = y[:, :_H]
    yit_ref[...] = y[:, _H:]


def _combine_mm(pp, cnt, b, xprev, w_cat, residual):
    return pl.pallas_call(
        functools.partial(_combine_body, residual),
        out_shape=[jax.ShapeDtypeStruct((_NP, _H), jnp.float32),
                   jax.ShapeDtypeStruct((_NP, _H), jnp.float32),
                   jax.ShapeDtypeStruct((_NP, _H), jnp.float32)],
    )(pp, cnt, b, xprev, w_cat)


def _tail_body(pp_ref, cnt_ref, b_ref, xprev_ref,
               wl_ref, bl_ref, wd1_ref, bd1_ref, wd2_ref, bd2_ref, out_ref):
    cnt = cnt_ref[0] + cnt_ref[1]
    inv = 1.0 / jnp.maximum(cnt[:, 0:1], 1.0)
    pp = pp_ref[0] + pp_ref[1]
    agg = pp[:, :_H] + pp[:, _H:] * inv + b_ref[...]
    x = jnp.maximum(agg, 0.0) + xprev_ref[...]
    feat = jnp.dot(x, wl_ref[...], preferred_element_type=jnp.float32)
    feat = feat + bl_ref[...]
    h = jnp.maximum(
        jnp.dot(feat, wd1_ref[...], preferred_element_type=jnp.float32)
        + bd1_ref[...], 0.0)
    logits = (jnp.dot(h, wd2_ref[...], preferred_element_type=jnp.float32)
              + bd2_ref[...])
    out_ref[...] = jax.nn.sigmoid(logits[:_N])


def _tail(pp, cnt, b, xprev, wl, bl, wd1, bd1, wd2, bd2):
    return pl.pallas_call(
        _tail_body,
        out_shape=jax.ShapeDtypeStruct((_N, 9), jnp.float32),
    )(pp, cnt, b, xprev, wl, bl, wd1, bd1, wd2, bd2)


def kernel(x_stroke, edge_index_temp_previous, edge_index_intersects,
           W_head_tp, W_head_int, b_head,
           W_tp1, W_int1, b1, W_tp2, W_int2, b2,
           W_tp3, W_int3, b3, W_tp4, W_int4, b4,
           Wl, bl, Wd1, bd1, Wd2, bd2):
    ei_tp = edge_index_temp_previous.reshape(2, _NW, _K, _C)
    ei_it = edge_index_intersects.reshape(2, _NW, _K, _C)

    zeros64 = jnp.zeros((_STRIPE, _H), jnp.float32)
    zeros16 = jnp.zeros((_STRIPE, _CW), jnp.float32)
    ones16 = jnp.ones((_C, _CW), jnp.float32)

    w_head_cat = jnp.concatenate([W_head_tp, W_head_int], axis=1)
    w_cats = [jnp.concatenate([wt, wi], axis=1)
              for (wt, wi) in ((W_tp1, W_int1), (W_tp2, W_int2),
                               (W_tp3, W_int3), (W_tp4, W_int4))]
    biases = [b_head.reshape(1, _H), b1.reshape(1, _H), b2.reshape(1, _H),
              b3.reshape(1, _H), b4.reshape(1, _H)]

    (cntp,) = _sc_count(ei_it, zeros16, ones16)

    y_tp, y_it = _head_mm(x_stroke, w_head_cat)

    (pp,) = _sc_scatter(y_tp, y_it, ei_tp, ei_it, zeros64)

    xprev = jnp.zeros((_NP, _H), jnp.float32)
    x, y_tp, y_it = _combine_mm(pp, cntp, biases[0], xprev,
                                w_cats[0], residual=False)

    for layer in range(1, 5):
        (pp,) = _sc_scatter(y_tp, y_it, ei_tp, ei_it, zeros64)
        if layer < 4:
            x, y_tp, y_it = _combine_mm(pp, cntp, biases[layer], x,
                                        w_cats[layer], residual=True)
        else:
            return _tail(pp, cntp, biases[4], x,
                         Wl, bl.reshape(1, 128), Wd1, bd1.reshape(1, _H),
                         Wd2, bd2.reshape(1, 9))


# speedup vs baseline: 4.3910x; 1.0389x over previous
import functools

import jax
import jax.numpy as jnp
from jax import lax
from jax.experimental import pallas as pl
from jax.experimental.pallas import tpu as pltpu
from jax.experimental.pallas import tpu_sc as plsc

_N = 10000
_NP = 10240
_E = 320000
_H = 64
_NC = 2
_NS = 16
_NW = _NC * _NS
_EW = _E // _NW
_C = 125
_K = 80
_EP = _NW * _K * _C
_STRIPE = _NP // _NS
_CW = 16
_NB = 4


def _sc_body(with_cnt, *refs):
    (y_tp, y_it, ei_tp, ei_it, zeros64,
     out_pp, acc,
     src_tp_v, dst_tp_v, src_it_v, dst_it_v, rows_v, sem) = refs
    c = lax.axis_index("c")
    s = lax.axis_index("s")
    wid = c * _NS + s
    stripe = s * _STRIPE

    pltpu.sync_copy(zeros64, acc.at[pl.ds(stripe, _STRIPE)])
    pltpu.sync_copy(ei_tp.at[0].at[wid], src_tp_v)
    pltpu.sync_copy(ei_tp.at[1].at[wid], dst_tp_v)
    pltpu.sync_copy(ei_it.at[0].at[wid], src_it_v)
    pltpu.sync_copy(ei_it.at[1].at[wid], dst_it_v)
    plsc.subcore_barrier()

    def run_type(y, src_v, dst_v):
        for b in range(_NB):
            pltpu.async_copy(y.at[src_v.at[b]], rows_v[b], sem)

        def group(g, prefetch):
            for b in range(_NB):
                j = g * _NB + b
                pltpu.make_async_copy(y.at[src_v.at[j]], rows_v[b],
                                      sem).wait()
                pltpu.sync_copy(rows_v[b], acc.at[dst_v.at[j]], add=True)
                if prefetch:
                    pltpu.async_copy(y.at[src_v.at[j + _NB]], rows_v[b], sem)

        lax.fori_loop(0, _K // _NB - 1, lambda g, _: (group(g, True), 0)[1],
                      0)
        group(_K // _NB - 1, False)

    run_type(y_tp, src_tp_v, dst_tp_v)
    plsc.subcore_barrier()
    pltpu.sync_copy(acc.at[pl.ds(stripe, _STRIPE)],
                    out_pp.at[c].at[pl.ds(stripe, _STRIPE), pl.ds(0, _H)])
    pltpu.sync_copy(zeros64, acc.at[pl.ds(stripe, _STRIPE)])
    plsc.subcore_barrier()

    run_type(y_it, src_it_v, dst_it_v)
    plsc.subcore_barrier()
    pltpu.sync_copy(acc.at[pl.ds(stripe, _STRIPE)],
                    out_pp.at[c].at[pl.ds(stripe, _STRIPE), pl.ds(_H, _H)])


def _cnt_body(ei_it, zeros16, ones16, out_cnt, acc_cnt, dst_it_v, ones_v):
    c = lax.axis_index("c")
    s = lax.axis_index("s")
    wid = c * _NS + s
    stripe = s * _STRIPE
    pltpu.sync_copy(zeros16, acc_cnt.at[pl.ds(stripe, _STRIPE)])
    pltpu.sync_copy(ei_it.at[1].at[wid], dst_it_v)
    pltpu.sync_copy(ones16, ones_v)
    plsc.subcore_barrier()

    def chunk(j, _):
        pltpu.sync_copy(ones_v, acc_cnt.at[dst_it_v.at[j]], add=True)
        return 0

    lax.fori_loop(0, _K, chunk, 0)
    plsc.subcore_barrier()
    pltpu.sync_copy(acc_cnt.at[pl.ds(stripe, _STRIPE)],
                    out_cnt.at[c].at[pl.ds(stripe, _STRIPE)])


def _make_kernels():
    mesh = plsc.VectorSubcoreMesh(core_axis_name="c", subcore_axis_name="s")
    params = pltpu.CompilerParams(use_tc_tiling_on_sc=False)
    scatter = pl.kernel(
        functools.partial(_sc_body, False),
        out_type=[jax.ShapeDtypeStruct((_NC, _NP, 2 * _H), jnp.float32)],
        mesh=mesh,
        scratch_types=[
            pltpu.VMEM_SHARED((_NP, _H), jnp.float32),
            pltpu.VMEM((_K, _C), jnp.int32),
            pltpu.VMEM((_K, _C), jnp.int32),
            pltpu.VMEM((_K, _C), jnp.int32),
            pltpu.VMEM((_K, _C), jnp.int32),
            [pltpu.VMEM((_C, _H), jnp.float32)
             for _ in range(_NB)],
            pltpu.SemaphoreType.DMA,
        ],
        compiler_params=params, name="sc_scatter")
    cnt = pl.kernel(
        _cnt_body,
        out_type=[jax.ShapeDtypeStruct((_NC, _NP, _CW), jnp.float32)],
        mesh=mesh,
        scratch_types=[
            pltpu.VMEM_SHARED((_NP, _CW), jnp.float32),
            pltpu.VMEM((_K, _C), jnp.int32),
            pltpu.VMEM((_C, _CW), jnp.float32),
        ],
        compiler_params=params, name="sc_count")
    return scatter, cnt


_sc_scatter, _sc_count = _make_kernels()


def _head_body(x_ref, w_ref, ytp_ref, yit_ref):
    y = jnp.dot(x_ref[...], w_ref[...], preferred_element_type=jnp.float32)
    ytp_ref[pl.ds(0, _N)] = y[:, :_H]
    yit_ref[pl.ds(0, _N)] = y[:, _H:]
    pad = jnp.zeros((_NP - _N, _H), jnp.float32)
    ytp_ref[pl.ds(_N, _NP - _N)] = pad
    yit_ref[pl.ds(_N, _NP - _N)] = pad


def _head_mm(x, w_cat):
    return pl.pallas_call(
        _head_body,
        out_shape=[jax.ShapeDtypeStruct((_NP, _H), jnp.float32),
                   jax.ShapeDtypeStruct((_NP, _H), jnp.float32)],
    )(x, w_cat)


def _combine0_body(pp_ref, cnt_ref, b_ref, x_ref):
    cnt = cnt_ref[0] + cnt_ref[1]
    inv = 1.0 / jnp.maximum(cnt[:, 0:1], 1.0)
    pp = pp_ref[0] + pp_ref[1]
    agg = pp[:, :_H] + pp[:, _H:] * inv + b_ref[...]
    x_ref[...] = jnp.maximum(agg, 0.0)


def _combine0(pp, cnt, b):
    return pl.pallas_call(
        _combine0_body,
        out_shape=jax.ShapeDtypeStruct((_NP, _H), jnp.float32),
    )(pp, cnt, b)


def _combine_body(pp_ref, cnt_ref, b_ref, xprev_ref, wtp_ref, wit_ref,
                  x_ref):
    cnt = cnt_ref[0] + cnt_ref[1]
    inv = 1.0 / jnp.maximum(cnt[:, 0:1], 1.0)
    pp = pp_ref[0] + pp_ref[1]
    agg = (jnp.dot(pp[:, :_H], wtp_ref[...],
                   preferred_element_type=jnp.float32)
           + jnp.dot(pp[:, _H:] * inv, wit_ref[...],
                     preferred_element_type=jnp.float32)
           + b_ref[...])
    x_ref[...] = jnp.maximum(agg, 0.0) + xprev_ref[...]


def _combine_mm(pp, cnt, b, xprev, wtp, wit):
    return pl.pallas_call(
        _combine_body,
        out_shape=jax.ShapeDtypeStruct((_NP, _H), jnp.float32),
    )(pp, cnt, b, xprev, wtp, wit)


def _tail_body(pp_ref, cnt_ref, b_ref, xprev_ref, wtp_ref, wit_ref,
               wl_ref, bl_ref, wd1_ref, bd1_ref, wd2_ref, bd2_ref, out_ref):
    cnt = cnt_ref[0] + cnt_ref[1]
    inv = 1.0 / jnp.maximum(cnt[:, 0:1], 1.0)
    pp = pp_ref[0] + pp_ref[1]
    agg = (jnp.dot(pp[:, :_H], wtp_ref[...],
                   preferred_element_type=jnp.float32)
           + jnp.dot(pp[:, _H:] * inv, wit_ref[...],
                     preferred_element_type=jnp.float32)
           + b_ref[...])
    x = jnp.maximum(agg, 0.0) + xprev_ref[...]
    feat = jnp.dot(x, wl_ref[...], preferred_element_type=jnp.float32)
    feat = feat + bl_ref[...]
    h = jnp.maximum(
        jnp.dot(feat, wd1_ref[...], preferred_element_type=jnp.float32)
        + bd1_ref[...], 0.0)
    logits = (jnp.dot(h, wd2_ref[...], preferred_element_type=jnp.float32)
              + bd2_ref[...])
    out_ref[...] = jax.nn.sigmoid(logits[:_N])


def _tail(pp, cnt, b, xprev, wtp, wit, wl, bl, wd1, bd1, wd2, bd2):
    return pl.pallas_call(
        _tail_body,
        out_shape=jax.ShapeDtypeStruct((_N, 9), jnp.float32),
    )(pp, cnt, b, xprev, wtp, wit, wl, bl, wd1, bd1, wd2, bd2)


def kernel(x_stroke, edge_index_temp_previous, edge_index_intersects,
           W_head_tp, W_head_int, b_head,
           W_tp1, W_int1, b1, W_tp2, W_int2, b2,
           W_tp3, W_int3, b3, W_tp4, W_int4, b4,
           Wl, bl, Wd1, bd1, Wd2, bd2):
    ei_tp = edge_index_temp_previous.reshape(2, _NW, _K, _C)
    ei_it = edge_index_intersects.reshape(2, _NW, _K, _C)

    zeros64 = jnp.zeros((_STRIPE, _H), jnp.float32)
    zeros16 = jnp.zeros((_STRIPE, _CW), jnp.float32)
    ones16 = jnp.ones((_C, _CW), jnp.float32)

    w_head_cat = jnp.concatenate([W_head_tp, W_head_int], axis=1)
    w_tps = [W_tp1, W_tp2, W_tp3, W_tp4]
    w_its = [W_int1, W_int2, W_int3, W_int4]
    biases = [b_head.reshape(1, _H), b1.reshape(1, _H), b2.reshape(1, _H),
              b3.reshape(1, _H), b4.reshape(1, _H)]

    (cntp,) = _sc_count(ei_it, zeros16, ones16)

    y_tp, y_it = _head_mm(x_stroke, w_head_cat)

    (pp,) = _sc_scatter(y_tp, y_it, ei_tp, ei_it, zeros64)

    x = _combine0(pp, cntp, biases[0])

    for layer in range(1, 5):
        (pp,) = _sc_scatter(x, x, ei_tp, ei_it, zeros64)
        if layer < 4:
            x = _combine_mm(pp, cntp, biases[layer], x,
                            w_tps[layer - 1], w_its[layer - 1])
        else:
            return _tail(pp, cntp, biases[4], x, w_tps[3], w_its[3],
                         Wl, bl.reshape(1, 128), Wd1, bd1.reshape(1, _H),
                         Wd2, bd2.reshape(1, 9))


# speedup vs baseline: 4.4416x; 1.0115x over previous
import functools

import jax
import jax.numpy as jnp
from jax import lax
from jax.experimental import pallas as pl
from jax.experimental.pallas import tpu as pltpu
from jax.experimental.pallas import tpu_sc as plsc

_N = 10000
_NP = 10240
_E = 320000
_H = 64
_NC = 2
_NS = 16
_NW = _NC * _NS
_EW = _E // _NW
_C = 125
_K = 80
_EP = _NW * _K * _C
_STRIPE = _NP // _NS
_CW = 16
_NB = 6


def _sc_body(with_cnt, *refs):
    (y_tp, y_it, ei_tp, ei_it, zeros64,
     out_pp, acc,
     src_tp_v, dst_tp_v, src_it_v, dst_it_v, rows_v, sem) = refs
    c = lax.axis_index("c")
    s = lax.axis_index("s")
    wid = c * _NS + s
    stripe = s * _STRIPE

    pltpu.sync_copy(zeros64, acc.at[pl.ds(stripe, _STRIPE)])
    pltpu.sync_copy(ei_tp.at[0].at[wid], src_tp_v)
    pltpu.sync_copy(ei_tp.at[1].at[wid], dst_tp_v)
    pltpu.sync_copy(ei_it.at[0].at[wid], src_it_v)
    pltpu.sync_copy(ei_it.at[1].at[wid], dst_it_v)
    plsc.subcore_barrier()

    def run_type(y, src_v, dst_v):
        for b in range(_NB):
            pltpu.async_copy(y.at[src_v.at[b]], rows_v[b], sem)

        def group(g, prefetch):
            for b in range(_NB):
                j = g * _NB + b
                pltpu.make_async_copy(y.at[src_v.at[j]], rows_v[b],
                                      sem).wait()
                pltpu.sync_copy(rows_v[b], acc.at[dst_v.at[j]], add=True)
                if prefetch:
                    pltpu.async_copy(y.at[src_v.at[j + _NB]], rows_v[b], sem)

        lax.fori_loop(0, _K // _NB - 1, lambda g, _: (group(g, True), 0)[1],
                      0)
        group(_K // _NB - 1, False)

    run_type(y_tp, src_tp_v, dst_tp_v)
    plsc.subcore_barrier()
    pltpu.sync_copy(acc.at[pl.ds(stripe, _STRIPE)],
                    out_pp.at[c].at[pl.ds(stripe, _STRIPE), pl.ds(0, _H)])
    pltpu.sync_copy(zeros64, acc.at[pl.ds(stripe, _STRIPE)])
    plsc.subcore_barrier()

    run_type(y_it, src_it_v, dst_it_v)
    plsc.subcore_barrier()
    pltpu.sync_copy(acc.at[pl.ds(stripe, _STRIPE)],
                    out_pp.at[c].at[pl.ds(stripe, _STRIPE), pl.ds(_H, _H)])


def _cnt_body(ei_it, zeros16, ones16, out_cnt, acc_cnt, dst_it_v, ones_v):
    c = lax.axis_index("c")
    s = lax.axis_index("s")
    wid = c * _NS + s
    stripe = s * _STRIPE
    pltpu.sync_copy(zeros16, acc_cnt.at[pl.ds(stripe, _STRIPE)])
    pltpu.sync_copy(ei_it.at[1].at[wid], dst_it_v)
    pltpu.sync_copy(ones16, ones_v)
    plsc.subcore_barrier()

    def chunk(j, _):
        pltpu.sync_copy(ones_v, acc_cnt.at[dst_it_v.at[j]], add=True)
        return 0

    lax.fori_loop(0, _K, chunk, 0)
    plsc.subcore_barrier()
    pltpu.sync_copy(acc_cnt.at[pl.ds(stripe, _STRIPE)],
                    out_cnt.at[c].at[pl.ds(stripe, _STRIPE)])


def _make_kernels():
    mesh = plsc.VectorSubcoreMesh(core_axis_name="c", subcore_axis_name="s")
    params = pltpu.CompilerParams(use_tc_tiling_on_sc=False)
    scatter = pl.kernel(
        functools.partial(_sc_body, False),
        out_type=[jax.ShapeDtypeStruct((_NC, _NP, 2 * _H), jnp.float32)],
        mesh=mesh,
        scratch_types=[
            pltpu.VMEM_SHARED((_NP, _H), jnp.float32),
            pltpu.VMEM((_K, _C), jnp.int32),
            pltpu.VMEM((_K, _C), jnp.int32),
            pltpu.VMEM((_K, _C), jnp.int32),
            pltpu.VMEM((_K, _C), jnp.int32),
            [pltpu.VMEM((_C, _H), jnp.float32)
             for _ in range(_NB)],
            pltpu.SemaphoreType.DMA,
        ],
        compiler_params=params, name="sc_scatter")
    cnt = pl.kernel(
        _cnt_body,
        out_type=[jax.ShapeDtypeStruct((_NC, _NP, _CW), jnp.float32)],
        mesh=mesh,
        scratch_types=[
            pltpu.VMEM_SHARED((_NP, _CW), jnp.float32),
            pltpu.VMEM((_K, _C), jnp.int32),
            pltpu.VMEM((_C, _CW), jnp.float32),
        ],
        compiler_params=params, name="sc_count")
    return scatter, cnt


_sc_scatter, _sc_count = _make_kernels()


def _head_body(x_ref, w_ref, ytp_ref, yit_ref):
    y = jnp.dot(x_ref[...], w_ref[...], preferred_element_type=jnp.float32)
    ytp_ref[pl.ds(0, _N)] = y[:, :_H]
    yit_ref[pl.ds(0, _N)] = y[:, _H:]
    pad = jnp.zeros((_NP - _N, _H), jnp.float32)
    ytp_ref[pl.ds(_N, _NP - _N)] = pad
    yit_ref[pl.ds(_N, _NP - _N)] = pad


def _head_mm(x, w_cat):
    return pl.pallas_call(
        _head_body,
        out_shape=[jax.ShapeDtypeStruct((_NP, _H), jnp.float32),
                   jax.ShapeDtypeStruct((_NP, _H), jnp.float32)],
    )(x, w_cat)


def _combine0_body(pp_ref, cnt_ref, b_ref, x_ref):
    cnt = cnt_ref[0] + cnt_ref[1]
    inv = 1.0 / jnp.maximum(cnt[:, 0:1], 1.0)
    pp = pp_ref[0] + pp_ref[1]
    agg = pp[:, :_H] + pp[:, _H:] * inv + b_ref[...]
    x_ref[...] = jnp.maximum(agg, 0.0)


def _combine0(pp, cnt, b):
    return pl.pallas_call(
        _combine0_body,
        out_shape=jax.ShapeDtypeStruct((_NP, _H), jnp.float32),
    )(pp, cnt, b)


def _combine_body(pp_ref, cnt_ref, b_ref, xprev_ref, wtp_ref, wit_ref,
                  x_ref):
    cnt = cnt_ref[0] + cnt_ref[1]
    inv = 1.0 / jnp.maximum(cnt[:, 0:1], 1.0)
    pp = pp_ref[0] + pp_ref[1]
    agg = (jnp.dot(pp[:, :_H], wtp_ref[...],
                   preferred_element_type=jnp.float32)
           + jnp.dot(pp[:, _H:] * inv, wit_ref[...],
                     preferred_element_type=jnp.float32)
           + b_ref[...])
    x_ref[...] = jnp.maximum(agg, 0.0) + xprev_ref[...]


def _combine_mm(pp, cnt, b, xprev, wtp, wit):
    return pl.pallas_call(
        _combine_body,
        out_shape=jax.ShapeDtypeStruct((_NP, _H), jnp.float32),
    )(pp, cnt, b, xprev, wtp, wit)


def _tail_body(pp_ref, cnt_ref, b_ref, xprev_ref, wtp_ref, wit_ref,
               wl_ref, bl_ref, wd1_ref, bd1_ref, wd2_ref, bd2_ref, out_ref):
    cnt = cnt_ref[0] + cnt_ref[1]
    inv = 1.0 / jnp.maximum(cnt[:, 0:1], 1.0)
    pp = pp_ref[0] + pp_ref[1]
    agg = (jnp.dot(pp[:, :_H], wtp_ref[...],
                   preferred_element_type=jnp.float32)
           + jnp.dot(pp[:, _H:] * inv, wit_ref[...],
                     preferred_element_type=jnp.float32)
           + b_ref[...])
    x = jnp.maximum(agg, 0.0) + xprev_ref[...]
    feat = jnp.dot(x, wl_ref[...], preferred_element_type=jnp.float32)
    feat = feat + bl_ref[...]
    h = jnp.maximum(
        jnp.dot(feat, wd1_ref[...], preferred_element_type=jnp.float32)
        + bd1_ref[...], 0.0)
    logits = (jnp.dot(h, wd2_ref[...], preferred_element_type=jnp.float32)
              + bd2_ref[...])
    out_ref[...] = jax.nn.sigmoid(logits[:_N])


def _tail(pp, cnt, b, xprev, wtp, wit, wl, bl, wd1, bd1, wd2, bd2):
    return pl.pallas_call(
        _tail_body,
        out_shape=jax.ShapeDtypeStruct((_N, 9), jnp.float32),
    )(pp, cnt, b, xprev, wtp, wit, wl, bl, wd1, bd1, wd2, bd2)


def kernel(x_stroke, edge_index_temp_previous, edge_index_intersects,
           W_head_tp, W_head_int, b_head,
           W_tp1, W_int1, b1, W_tp2, W_int2, b2,
           W_tp3, W_int3, b3, W_tp4, W_int4, b4,
           Wl, bl, Wd1, bd1, Wd2, bd2):
    ei_tp = edge_index_temp_previous.reshape(2, _NW, _K, _C)
    ei_it = edge_index_intersects.reshape(2, _NW, _K, _C)

    zeros64 = jnp.zeros((_STRIPE, _H), jnp.float32)
    zeros16 = jnp.zeros((_STRIPE, _CW), jnp.float32)
    ones16 = jnp.ones((_C, _CW), jnp.float32)

    w_head_cat = jnp.concatenate([W_head_tp, W_head_int], axis=1)
    w_tps = [W_tp1, W_tp2, W_tp3, W_tp4]
    w_its = [W_int1, W_int2, W_int3, W_int4]
    biases = [b_head.reshape(1, _H), b1.reshape(1, _H), b2.reshape(1, _H),
              b3.reshape(1, _H), b4.reshape(1, _H)]

    (cntp,) = _sc_count(ei_it, zeros16, ones16)

    y_tp, y_it = _head_mm(x_stroke, w_head_cat)

    (pp,) = _sc_scatter(y_tp, y_it, ei_tp, ei_it, zeros64)

    x = _combine0(pp, cntp, biases[0])

    for layer in range(1, 5):
        (pp,) = _sc_scatter(x, x, ei_tp, ei_it, zeros64)
        if layer < 4:
            x = _combine_mm(pp, cntp, biases[layer], x,
                            w_tps[layer - 1], w_its[layer - 1])
        else:
            return _tail(pp, cntp, biases[4], x, w_tps[3], w_its[3],
                         Wl, bl.reshape(1, 128), Wd1, bd1.reshape(1, _H),
                         Wd2, bd2.reshape(1, 9))
